# baseline (speedup 1.0000x reference)
import jax
import jax.numpy as jnp
from jax import lax
from jax.experimental import pallas as pl
from jax.experimental.pallas import tpu as pltpu
from jax.experimental.pallas import tpu_sc as plsc

_N = 50000
_L = 3
_NSC = 2
_NSUB = 16
_KEDGE = 128
_DUMMY = 48
_EALIGN = _NSC * _NSUB * _KEDGE
_NP = _N + _DUMMY
_ZR = _NP // _NSUB


def _pad_edges(e):
    return ((e + _EALIGN - 1) // _EALIGN) * _EALIGN


def _chunk_pass(s, bufs, tab_ref, src_ref, dst_ref, out_ref, zeros_ref,
                n_src, e_pad, q, split_scs):
    srcv, dstv, rows, acc, si, sg, ss, ones, c = bufs
    nworker = _NSUB * (_NSC if split_scs else 1)
    ept = e_pad // nworker
    nblk = ept // _KEDGE
    off = q * n_src
    w = (c * _NSUB + s) if split_scs else s
    gather = tab_ref is not None

    def src_sl(i):
        return src_ref.at[pl.ds(w * ept + i * _KEDGE, _KEDGE)]

    def dst_sl(i):
        return dst_ref.at[pl.ds(w * ept + i * _KEDGE, _KEDGE)]

    def start_idx(i, b):
        if gather:
            pltpu.async_copy(src_sl(i), srcv[b], si[b])
        pltpu.async_copy(dst_sl(i), dstv[b], si[b])

    def wait_idx(i, b):
        if gather:
            pltpu.make_async_copy(src_sl(i), srcv[b], si[b]).wait()
            if off:
                for k in range(_KEDGE // 16):
                    sl = pl.ds(k * 16, 16)
                    srcv[b][sl] = srcv[b][sl] + jnp.full((16,), off,
                                                         jnp.int32)
        pltpu.make_async_copy(dst_sl(i), dstv[b], si[b]).wait()

    def start_gather(b):
        pltpu.async_copy(tab_ref.at[srcv[b]], rows[b], sg[b])

    def wait_gather(b):
        pltpu.make_async_copy(tab_ref.at[srcv[b]], rows[b], sg[b]).wait()

    def start_scatter(b):
        pltpu.async_copy(rows[b] if gather else ones, acc.at[dstv[b]],
                         ss[b], add=True)

    def wait_scatter(b):
        pltpu.make_async_copy(rows[b] if gather else ones, acc.at[dstv[b]],
                              ss[b]).wait()

    pltpu.sync_copy(zeros_ref.at[pl.ds(s * _ZR, _ZR)],
                    acc.at[pl.ds(s * _ZR, _ZR)])
    plsc.subcore_barrier()

    def step(i, b, static):
        if gather:
            wait_gather(b)
        else:
            wait_idx(i, b)
        start_scatter(b)
        b1, b2 = (b + 1) % 3, (b + 2) % 3

        def do_ws():
            wait_scatter(b2)

        def do_idx():
            start_idx(i + 2, b2)

        def do_g():
            wait_idx(i + 1, b1)
            start_gather(b1)

        if static:
            if i > 0:
                do_ws()
            if i + 2 < nblk:
                do_idx()
            if gather and i + 1 < nblk:
                do_g()
        else:
            pl.when(i > 0)(do_ws)
            pl.when(i + 2 < nblk)(do_idx)
            if gather:
                pl.when(i + 1 < nblk)(do_g)

    start_idx(0, 0)
    if nblk > 1:
        start_idx(1, 1)
    if gather:
        wait_idx(0, 0)
        start_gather(0)
    nsup, rem = nblk // 3, nblk % 3

    def sup_body(sup, carry):
        for j in range(3):
            step(sup * 3 + j, j, False)
        return carry

    lax.fori_loop(0, nsup, sup_body, 0)
    for j in range(rem):
        step(nsup * 3 + j, j, True)
    wait_scatter((nblk - 1) % 3)
    plsc.subcore_barrier()
    qout = (q * _NSC + c) if split_scs else q
    pltpu.sync_copy(acc.at[pl.ds(s * _ZR, _ZR)],
                    out_ref.at[pl.ds(qout * _NP + s * _ZR, _ZR)])


_SC_PARAMS = pltpu.CompilerParams(use_tc_tiling_on_sc=False)
_SC_MESH = plsc.VectorSubcoreMesh(core_axis_name="c", subcore_axis_name="s")


def _sc_scratch(cw, with_ones):
    return (
        [pltpu.VMEM((_KEDGE,), jnp.int32) for _ in range(6)]
        + [pltpu.VMEM((_KEDGE, cw), jnp.float32) for _ in range(3)]
        + [pltpu.VMEM_SHARED((_NP, cw), jnp.float32)]
        + [pltpu.SemaphoreType.DMA for _ in range(9)]
        + ([pltpu.VMEM((_KEDGE, cw), jnp.float32)] if with_ones else [])
    )


def _mk_bufs(scr, c):
    return (scr[0:3], scr[3:6], scr[6:9], scr[9], scr[10:13], scr[13:16],
            scr[16:19], scr[19] if len(scr) > 19 else None, c)


def _make_spatial(e_pad):
    def body(tab_ref, src_ref, dst_ref, zeros_ref, out_ref, *scr):
        c = lax.axis_index("c")
        s = lax.axis_index("s")
        bufs = _mk_bufs(scr, c)

        def sc0():
            _chunk_pass(s, bufs, tab_ref, src_ref, dst_ref, out_ref,
                        zeros_ref, _N, e_pad, 0, False)
            _chunk_pass(s, bufs, tab_ref, src_ref, dst_ref, out_ref,
                        zeros_ref, _N, e_pad, 1, False)

        def sc1():
            _chunk_pass(s, bufs, tab_ref, src_ref, dst_ref, out_ref,
                        zeros_ref, _N, e_pad, 2, False)
            _chunk_pass(s, bufs, tab_ref, src_ref, dst_ref, out_ref,
                        zeros_ref, _N, e_pad, 3, False)

        pl.when(c == 0)(sc0)
        pl.when(c == 1)(sc1)

    return pl.kernel(
        body,
        out_type=jax.ShapeDtypeStruct((4 * _NP, 32), jnp.float32),
        mesh=_SC_MESH,
        scratch_types=_sc_scratch(32, False),
        compiler_params=_SC_PARAMS,
    )


def _make_static_aggs(e_l, e_s, e_i):
    def body(tab_l, src_l, dst_l, tab_s, src_s, dst_s, tab_i, src_i, dst_i,
             zeros_ref, out_l, out_s, out_i, *scr):
        c = lax.axis_index("c")
        s = lax.axis_index("s")
        bufs = _mk_bufs(scr, c)

        def pass_l(q):
            _chunk_pass(s, bufs, tab_l, src_l, dst_l, out_l, zeros_ref,
                        50000, e_l, q, False)

        def pass_s(q):
            _chunk_pass(s, bufs, tab_s, src_s, dst_s, out_s, zeros_ref,
                        20000, e_s, q, False)

        def pass_i(q):
            _chunk_pass(s, bufs, tab_i, src_i, dst_i, out_i, zeros_ref,
                        10000, e_i, q, False)

        def sc0():
            pass_l(0)
            pass_l(1)
            pass_s(0)

        def sc1():
            pass_l(2)
            pass_l(3)
            pass_s(1)
            pass_i(0)

        pl.when(c == 0)(sc0)
        pl.when(c == 1)(sc1)

    return pl.kernel(
        body,
        out_type=[
            jax.ShapeDtypeStruct((4 * _NP, 16), jnp.float32),
            jax.ShapeDtypeStruct((2 * _NP, 16), jnp.float32),
            jax.ShapeDtypeStruct((1 * _NP, 16), jnp.float32),
        ],
        mesh=_SC_MESH,
        scratch_types=_sc_scratch(16, False),
        compiler_params=_SC_PARAMS,
    )


def _make_counts_all(e_sp, e_l, e_s, e_i):
    def body(d_sp, d_l, d_s, d_i, zeros_ref, o_sp, o_l, o_s, o_i, *scr):
        c = lax.axis_index("c")
        s = lax.axis_index("s")
        bufs = _mk_bufs(scr, c)
        ones = bufs[7]
        for i in range(_KEDGE):
            ones[i, :] = jnp.full((16,), 1.0, jnp.float32)
        for dref, oref, ep in ((d_sp, o_sp, e_sp), (d_l, o_l, e_l),
                               (d_s, o_s, e_s), (d_i, o_i, e_i)):
            _chunk_pass(s, bufs, None, dref, dref, oref, zeros_ref,
                        _N, ep, 0, True)

    return pl.kernel(
        body,
        out_type=[jax.ShapeDtypeStruct((_NSC * _NP, 16), jnp.float32)
                  for _ in range(4)],
        mesh=_SC_MESH,
        scratch_types=_sc_scratch(16, True),
        compiler_params=_SC_PARAMS,
    )


_BM = 1000


def _proj_int_body(x_ref, w_ref, b_ref, out_ref, outq_ref):
    y = jnp.dot(x_ref[...], w_ref[...],
                preferred_element_type=jnp.float32) + b_ref[...]
    out_ref[...] = y
    for q in range(4):
        outq_ref[q] = y[:, 32 * q:32 * (q + 1)]


def _proj_int(x, w, b):
    nb = x.shape[0] // _BM
    return pl.pallas_call(
        _proj_int_body,
        grid=(nb,),
        in_specs=[
            pl.BlockSpec((_BM, 128), lambda i: (i, 0)),
            pl.BlockSpec((128, 128), lambda i: (0, 0)),
            pl.BlockSpec((1, 128), lambda i: (0, 0)),
        ],
        out_specs=[
            pl.BlockSpec((_BM, 128), lambda i: (i, 0)),
            pl.BlockSpec((4, _BM, 32), lambda i: (0, i, 0)),
        ],
        out_shape=[
            jax.ShapeDtypeStruct((x.shape[0], 128), jnp.float32),
            jax.ShapeDtypeStruct((4, x.shape[0], 32), jnp.float32),
        ],
    )(x, w, b)


def _csum(cnt_blk):
    return cnt_blk[0, :, 0:1] + cnt_blk[1, :, 0:1]


def _const_build_body(al_ref, as_ref, ai_ref, cl_ref, cs_ref, ci_ref,
                      wls_ref, winj_ref, bl_ref, bs_ref, bi_ref, wfwi_ref,
                      out_ref):
    cl = _csum(cl_ref[...])
    cs = _csum(cs_ref[...])
    ci = _csum(ci_ref[...])
    rl = 1.0 / jnp.maximum(cl, 1.0)
    rs = 1.0 / jnp.maximum(cs, 1.0)
    ri = 1.0 / jnp.maximum(ci, 1.0)
    albk = al_ref[...]
    asbk = as_ref[...]
    ml = jnp.concatenate([albk[q] for q in range(4)], axis=1) * rl
    ms = jnp.concatenate([asbk[q] for q in range(2)], axis=1) * rs
    mi = ai_ref[0] * ri
    u = jnp.dot(jnp.concatenate([ml, ms], axis=1), wls_ref[...],
                preferred_element_type=jnp.float32)
    u = u + jnp.where(cl > 0, 1.0, 0.0) * bl_ref[...]
    u = u + jnp.where(cs > 0, 1.0, 0.0) * bs_ref[...]
    v = jnp.dot(mi, winj_ref[...], preferred_element_type=jnp.float32)
    v = v + jnp.where(ci > 0, 1.0, 0.0) * bi_ref[...]
    out_ref[...] = jnp.dot(jnp.concatenate([u, v], axis=1), wfwi_ref[...],
                           preferred_element_type=jnp.float32)


def _const_build(agg_lane, agg_sens, agg_inj, cnt_fl, cnt_fs, cnt_inc,
                 w_ls, wp_inj, b_lane, b_sens, b_inj, wfwi):
    nb = _N // _BM
    return pl.pallas_call(
        _const_build_body,
        grid=(nb,),
        in_specs=[
            pl.BlockSpec((4, _BM, 16), lambda i: (0, i, 0)),
            pl.BlockSpec((2, _BM, 16), lambda i: (0, i, 0)),
            pl.BlockSpec((1, _BM, 16), lambda i: (0, i, 0)),
            pl.BlockSpec((2, _BM, 16), lambda i: (0, i, 0)),
            pl.BlockSpec((2, _BM, 16), lambda i: (0, i, 0)),
            pl.BlockSpec((2, _BM, 16), lambda i: (0, i, 0)),
            pl.BlockSpec((96, 128), lambda i: (0, 0)),
            pl.BlockSpec((16, 128), lambda i: (0, 0)),
            pl.BlockSpec((1, 128), lambda i: (0, 0)),
            pl.BlockSpec((1, 128), lambda i: (0, 0)),
            pl.BlockSpec((1, 128), lambda i: (0, 0)),
            pl.BlockSpec((256, 384), lambda i: (0, 0)),
        ],
        out_specs=pl.BlockSpec((_BM, 384), lambda i: (i, 0)),
        out_shape=jax.ShapeDtypeStruct((_N, 384), jnp.float32),
    )(agg_lane, agg_sens, agg_inj, cnt_fl, cnt_fs, cnt_inc,
      w_ls, wp_inj, b_lane, b_sens, b_inj, wfwi)


def _combine_body(h_ref, sp_ref, csp_ref, const_ref, ws_ref, wsp_ref, b_ref,
                  out_ref, outq_ref):
    csp = _csum(csp_ref[...])
    rsp = 1.0 / jnp.maximum(csp, 1.0)
    spbk = sp_ref[...]
    msp = jnp.concatenate([spbk[q] for q in range(4)], axis=1) * rsp
    y = jnp.dot(h_ref[...], ws_ref[...], preferred_element_type=jnp.float32)
    y = y + jnp.dot(msp, wsp_ref[...], preferred_element_type=jnp.float32)
    y = y + b_ref[...] + const_ref[...]
    y = jnp.where(y > 0, y, jnp.exp(jnp.minimum(y, 0.0)) - 1.0)
    out_ref[...] = y
    for q in range(4):
        outq_ref[q] = y[:, 32 * q:32 * (q + 1)]


def _combine(h, agg_sp, cnt_sp, const_all, w_self_l, w_sp_l, b_l, lidx):
    nb = _N // _BM
    return pl.pallas_call(
        _combine_body,
        grid=(nb,),
        in_specs=[
            pl.BlockSpec((_BM, 128), lambda i: (i, 0)),
            pl.BlockSpec((4, _BM, 32), lambda i: (0, i, 0)),
            pl.BlockSpec((2, _BM, 16), lambda i: (0, i, 0)),
            pl.BlockSpec((_BM, 128), lambda i, _l=lidx: (i, _l)),
            pl.BlockSpec((128, 128), lambda i: (0, 0)),
            pl.BlockSpec((128, 128), lambda i: (0, 0)),
            pl.BlockSpec((1, 128), lambda i: (0, 0)),
        ],
        out_specs=[
            pl.BlockSpec((_BM, 128), lambda i: (i, 0)),
            pl.BlockSpec((4, _BM, 32), lambda i: (0, i, 0)),
        ],
        out_shape=[
            jax.ShapeDtypeStruct((_N, 128), jnp.float32),
            jax.ShapeDtypeStruct((4, _N, 32), jnp.float32),
        ],
    )(h, agg_sp, cnt_sp, const_all, w_self_l, w_sp_l, b_l)


def _prep_edges(edges):
    src, dst = edges[0], edges[1]
    e = src.shape[0]
    e_pad = _pad_edges(e)
    pad = e_pad - e
    src_p = jnp.concatenate([src, jnp.zeros((pad,), jnp.int32)])
    dst_p = jnp.concatenate(
        [dst, _N + (jnp.arange(pad, dtype=jnp.int32) % _DUMMY)])
    return src_p, dst_p, e_pad


def _chunk16(x, nch):
    n = x.shape[0]
    return jnp.transpose(x.reshape(n, nch, 16), (1, 0, 2)).reshape(nch * n, 16)


def kernel(x_int, x_lane, x_sens, x_inj, Wp_int, bp_int, Wp_lane, bp_lane,
           Wp_sens, bp_sens, Wp_inj, bp_inj, W_self, b_self, W_spatial,
           W_flow, W_incident, spatial_e, flow_lane_e, flow_sens_e,
           incident_e):
    f32 = jnp.float32
    zeros16 = jnp.zeros((_NP, 16), f32)
    zeros32 = jnp.zeros((_NP, 32), f32)

    sp_src, sp_dst, sp_ep = _prep_edges(spatial_e)
    fl_src, fl_dst, fl_ep = _prep_edges(flow_lane_e)
    fs_src, fs_dst, fs_ep = _prep_edges(flow_sens_e)
    inc_src, inc_dst, inc_ep = _prep_edges(incident_e)

    cnt_sp, cnt_fl, cnt_fs, cnt_inc = _make_counts_all(
        sp_ep, fl_ep, fs_ep, inc_ep)(sp_dst, fl_dst, fs_dst, inc_dst,
                                     zeros16)

    agg_lane, agg_sens, agg_inj = _make_static_aggs(fl_ep, fs_ep, inc_ep)(
        _chunk16(x_lane, 4), fl_src, fl_dst,
        _chunk16(x_sens, 2), fs_src, fs_dst,
        _chunk16(x_inj, 1), inc_src, inc_dst, zeros16)

    h, hq = _proj_int(x_int, Wp_int, bp_int.reshape(1, 128))

    w_ls = jnp.concatenate([Wp_lane, Wp_sens], axis=0)
    wf3 = jnp.transpose(W_flow, (1, 0, 2)).reshape(128, 384)
    wi3 = jnp.transpose(W_incident, (1, 0, 2)).reshape(128, 384)
    wfwi = jnp.concatenate([wf3, wi3], axis=0)
    const_all = _const_build(
        agg_lane.reshape(4, _NP, 16), agg_sens.reshape(2, _NP, 16),
        agg_inj.reshape(1, _NP, 16),
        cnt_fl.reshape(2, _NP, 16), cnt_fs.reshape(2, _NP, 16),
        cnt_inc.reshape(2, _NP, 16),
        w_ls, Wp_inj, bp_lane.reshape(1, 128), bp_sens.reshape(1, 128),
        bp_inj.reshape(1, 128), wfwi)

    cnt_sp_r = cnt_sp.reshape(2, _NP, 16)
    seg_sp = _make_spatial(sp_ep)
    for l in range(_L):
        agg_sp = seg_sp(hq.reshape(4 * _N, 32), sp_src, sp_dst, zeros32)
        h, hq = _combine(h, agg_sp.reshape(4, _NP, 32), cnt_sp_r, const_all,
                         W_self[l], W_spatial[l], b_self[l].reshape(1, 128),
                         l)
    return h
